# Initial kernel scaffold; baseline (speedup 1.0000x reference)
import jax
import jax.numpy as jnp
from jax.experimental import pallas as pl


def kernel(x, batch, codebook, causal_codebook, counter_codebook, W, b):
    raise NotImplementedError("write your pallas kernel here")


# speedup vs baseline: 2.7597x; 2.7597x over previous
import functools

import jax
import jax.numpy as jnp
from jax import lax
from jax.experimental import pallas as pl
from jax.experimental.pallas import tpu as pltpu
from jax.experimental.pallas import tpu_sc as plsc

NSEG = 256


def _argmin_body(nk, bk, x_ref, cc_ref, b_ref, idx_ref, sumx_ref, cnt_ref,
                 minval, minidx):
    j = pl.program_id(1)
    x = x_ref[...]
    bn = x.shape[0]

    @pl.when(jnp.logical_and(pl.program_id(0) == 0, j == 0))
    def _():
        sumx_ref[...] = jnp.zeros_like(sumx_ref)
        cnt_ref[...] = jnp.zeros_like(cnt_ref)

    @pl.when(j == 0)
    def _():
        b2 = b_ref[0]
        seg = lax.broadcasted_iota(jnp.int32, (bn, NSEG), 1)
        mask_t = (b2 == seg).astype(jnp.float32)
        sumx_ref[...] += lax.dot_general(
            mask_t, x, (((0,), (0,)), ((), ())),
            preferred_element_type=jnp.float32)
        cnt_ref[...] += lax.dot_general(
            mask_t, jnp.ones((bn, 1), jnp.float32), (((0,), (0,)), ((), ())),
            preferred_element_type=jnp.float32)

    cc = cc_ref[...]
    c2 = jnp.sum(cc * cc, axis=1)
    x2 = jnp.sum(x * x, axis=1, keepdims=True)
    mm = lax.dot_general(x, cc, (((1,), (1,)), ((), ())),
                         preferred_element_type=jnp.float32)
    scores = x2 + c2[None, :] - 2.0 * mm
    bm = jnp.min(scores, axis=1, keepdims=True)
    col = lax.broadcasted_iota(jnp.int32, scores.shape, 1)
    bidx = jnp.min(jnp.where(scores == bm, col, jnp.int32(2 ** 30)),
                   axis=1, keepdims=True) + j * bk

    @pl.when(j == 0)
    def _():
        minval[...] = bm
        minidx[...] = bidx

    @pl.when(j > 0)
    def _():
        better = bm < minval[...]
        minval[...] = jnp.where(better, bm, minval[...])
        minidx[...] = jnp.where(better, bidx, minidx[...])

    @pl.when(j == nk - 1)
    def _():
        idx_ref[0] = minidx[...]


def _argmin_call(x, cc, batch, bn, bk):
    n, d = x.shape
    k = cc.shape[0]
    nn, nk = n // bn, k // bk
    b3 = batch.reshape(nn, bn, 1)
    return pl.pallas_call(
        functools.partial(_argmin_body, nk, bk),
        grid=(nn, nk),
        in_specs=[
            pl.BlockSpec((bn, d), lambda i, j: (i, 0)),
            pl.BlockSpec((bk, d), lambda i, j: (j, 0)),
            pl.BlockSpec((1, bn, 1), lambda i, j: (i, 0, 0)),
        ],
        out_specs=[
            pl.BlockSpec((1, bn, 1), lambda i, j: (i, 0, 0)),
            pl.BlockSpec((NSEG, d), lambda i, j: (0, 0)),
            pl.BlockSpec((NSEG, 1), lambda i, j: (0, 0)),
        ],
        out_shape=[
            jax.ShapeDtypeStruct((nn, bn, 1), jnp.int32),
            jax.ShapeDtypeStruct((NSEG, d), jnp.float32),
            jax.ShapeDtypeStruct((NSEG, 1), jnp.float32),
        ],
        scratch_shapes=[
            pltpu.VMEM((bn, 1), jnp.float32),
            pltpu.VMEM((bn, 1), jnp.int32),
        ],
    )(x, cc, b3)


def _sc_gather3(idx, t0, t1, t2):
    n = idx.shape[0]
    d = t0.shape[1]
    nw = 32
    rw = 320
    ch = 80
    assert n == 31 * rw + ch and rw % ch == 0

    mesh = plsc.VectorSubcoreMesh(core_axis_name="c", subcore_axis_name="s",
                                  num_cores=2, num_subcores=16)
    out_t = [jax.ShapeDtypeStruct((n, d), jnp.float32)] * 3

    @functools.partial(
        pl.kernel, out_type=out_t, mesh=mesh,
        scratch_types=[
            pltpu.VMEM((ch,), jnp.int32),
            pltpu.VMEM((ch, d), jnp.float32),
            pltpu.VMEM((ch, d), jnp.float32),
            pltpu.VMEM((ch, d), jnp.float32),
            pltpu.SemaphoreType.DMA,
            pltpu.SemaphoreType.DMA,
            pltpu.SemaphoreType.DMA,
        ],
    )
    def k(idx_hbm, t0_hbm, t1_hbm, t2_hbm, o0_hbm, o1_hbm, o2_hbm,
          idx_v, r0, r1, r2, s0, s1, s2):
        wid = lax.axis_index("s") * 2 + lax.axis_index("c")
        base = wid * rw
        for c in range(rw // ch):
            off = jnp.minimum(base + c * ch, n - ch)
            pltpu.sync_copy(idx_hbm.at[pl.ds(off, ch)], idx_v)
            cp0 = pltpu.make_async_copy(t0_hbm.at[idx_v], r0, s0)
            cp1 = pltpu.make_async_copy(t1_hbm.at[idx_v], r1, s1)
            cp2 = pltpu.make_async_copy(t2_hbm.at[idx_v], r2, s2)
            cp0.start(); cp1.start(); cp2.start()
            cp0.wait(); cp1.wait(); cp2.wait()
            pltpu.sync_copy(r0, o0_hbm.at[pl.ds(off, ch)])
            pltpu.sync_copy(r1, o1_hbm.at[pl.ds(off, ch)])
            pltpu.sync_copy(r2, o2_hbm.at[pl.ds(off, ch)])

    return k(idx, t0, t1, t2)


def _pool_body(nn, causal_ref, counter_ref, b_ref, sumx_ref, cnt_ref,
               w_ref, bias_ref,
               cpre_ref, kpre_ref, ypre_ref, pc_ref, px_ref,
               acc_c, acc_k):
    i = pl.program_id(0)
    b2 = b_ref[0]
    bn = b2.shape[0]
    seg = lax.broadcasted_iota(jnp.int32, (bn, NSEG), 1)
    mask_t = (b2 == seg).astype(jnp.float32)

    @pl.when(i == 0)
    def _():
        acc_c[...] = jnp.zeros_like(acc_c)
        acc_k[...] = jnp.zeros_like(acc_k)

    acc_c[...] += lax.dot_general(mask_t, causal_ref[...],
                                  (((0,), (0,)), ((), ())),
                                  preferred_element_type=jnp.float32)
    acc_k[...] += lax.dot_general(mask_t, counter_ref[...],
                                  (((0,), (0,)), ((), ())),
                                  preferred_element_type=jnp.float32)

    @pl.when(i == nn - 1)
    def _():
        cnt = jnp.maximum(cnt_ref[...], 1.0)
        pooled_x = sumx_ref[...] / cnt
        pooled_c = pooled_x + acc_c[...] / cnt
        pooled_k = acc_k[...] / cnt
        w = w_ref[...]
        bias = bias_ref[...]
        dn = (((1,), (1,)), ((), ()))
        cpre_ref[...] = lax.dot_general(
            pooled_c, w, dn, preferred_element_type=jnp.float32) + bias
        kpre_ref[...] = lax.dot_general(
            pooled_k, w, dn, preferred_element_type=jnp.float32) + bias
        ypre_ref[...] = lax.dot_general(
            pooled_x, w, dn, preferred_element_type=jnp.float32) + bias
        pc_ref[...] = pooled_c
        px_ref[...] = pooled_x


def _pool_call(causal_rows, counter_rows, batch, sumx, cnt, w, bias, bn):
    n, d = causal_rows.shape
    t = w.shape[0]
    nn = n // bn
    b3 = batch.reshape(nn, bn, 1)
    whole = lambda shape: pl.BlockSpec(shape, lambda i: tuple(0 for _ in shape))
    return pl.pallas_call(
        functools.partial(_pool_body, nn),
        grid=(nn,),
        in_specs=[
            pl.BlockSpec((bn, d), lambda i: (i, 0)),
            pl.BlockSpec((bn, d), lambda i: (i, 0)),
            pl.BlockSpec((1, bn, 1), lambda i: (i, 0, 0)),
            whole((NSEG, d)),
            whole((NSEG, 1)),
            whole((t, d)),
            whole((1, t)),
        ],
        out_specs=[
            whole((NSEG, t)),
            whole((NSEG, t)),
            whole((NSEG, t)),
            whole((NSEG, d)),
            whole((NSEG, d)),
        ],
        out_shape=[
            jax.ShapeDtypeStruct((NSEG, t), jnp.float32),
            jax.ShapeDtypeStruct((NSEG, t), jnp.float32),
            jax.ShapeDtypeStruct((NSEG, t), jnp.float32),
            jax.ShapeDtypeStruct((NSEG, d), jnp.float32),
            jax.ShapeDtypeStruct((NSEG, d), jnp.float32),
        ],
        scratch_shapes=[
            pltpu.VMEM((NSEG, d), jnp.float32),
            pltpu.VMEM((NSEG, d), jnp.float32),
        ],
    )(causal_rows, counter_rows, b3, sumx, cnt, w, bias)


def kernel(x, batch, codebook, causal_codebook, counter_codebook, W, b):
    n, d = x.shape
    batch = batch.astype(jnp.int32)
    bn = 1000
    bk = 1024

    idx3, sumx, cnt = _argmin_call(x, causal_codebook, batch, bn, bk)
    idx = idx3.reshape(n)

    z_nodes, causal_rows, counter_rows = _sc_gather3(
        idx, codebook, causal_codebook, counter_codebook)

    causal_pre, counter_pre, y_pre, pooled_causal, pooled_x = _pool_call(
        causal_rows, counter_rows, batch, sumx, cnt, W,
        b.reshape(1, -1), bn)

    return (causal_pre, counter_pre, y_pre, z_nodes, pooled_causal, pooled_x)
